# Initial kernel scaffold; baseline (speedup 1.0000x reference)
import jax
import jax.numpy as jnp
from jax.experimental import pallas as pl


def kernel(x, edge_index, edge_attr, W0, a_src0, a_dst0, b0, W1, a_src1, a_dst1, b1):
    raise NotImplementedError("write your pallas kernel here")


# speedup vs baseline: 12.8570x; 12.8570x over previous
import functools

import jax
import jax.numpy as jnp
from jax import lax
from jax.experimental import pallas as pl
from jax.experimental.pallas import tpu as pltpu
from jax.experimental.pallas import tpu_sc as plsc

N = 10000
NP = 10240
D = 128
H = 6
E = 320000
ETOT = E + N
NTILE = 16
CHUNK = 64
EPT = 20672
EP = NTILE * EPT
NCHUNK = EPT // CHUNK
NPT = NP // NTILE

_f32 = jnp.float32
_i32 = jnp.int32


def _mm_body(x_ref, w_ref, a_ref, h_ref, al_ref):
    h = jnp.dot(x_ref[...], w_ref[...], preferred_element_type=_f32)
    h_ref[...] = h
    al_ref[...] = jnp.dot(h, a_ref[...], preferred_element_type=_f32)


def _mm1(x, W, A):
    return pl.pallas_call(
        _mm_body,
        grid=(NP // 256,),
        in_specs=[
            pl.BlockSpec((256, D), lambda i: (i, 0)),
            pl.BlockSpec((D, H * D), lambda i: (0, 0)),
            pl.BlockSpec((H * D, D), lambda i: (0, 0)),
        ],
        out_specs=[
            pl.BlockSpec((256, H * D), lambda i: (i, 0)),
            pl.BlockSpec((256, D), lambda i: (i, 0)),
        ],
        out_shape=[
            jax.ShapeDtypeStruct((NP, H * D), _f32),
            jax.ShapeDtypeStruct((NP, D), _f32),
        ],
    )(x, W, A)


def _combine(p_ref, b_ref):
    acc = p_ref[0, 0]
    for c in range(2):
        for hh in range(3):
            if (c, hh) != (0, 0):
                acc = acc + p_ref[c, hh]
    return jnp.maximum(acc * (1.0 / H) + b_ref[...], 0.0)


def _mm2_body(p_ref, b_ref, w_ref, a_ref, h_ref, al_ref):
    x = _combine(p_ref, b_ref)
    h = jnp.dot(x, w_ref[...], preferred_element_type=_f32)
    h_ref[...] = h
    al_ref[...] = jnp.dot(h, a_ref[...], preferred_element_type=_f32)


def _mm2(part, b, W, A):
    return pl.pallas_call(
        _mm2_body,
        grid=(NP // 256,),
        in_specs=[
            pl.BlockSpec((2, 3, 256, D), lambda i: (0, 0, i, 0)),
            pl.BlockSpec((1, D), lambda i: (0, 0)),
            pl.BlockSpec((D, H * D), lambda i: (0, 0)),
            pl.BlockSpec((H * D, D), lambda i: (0, 0)),
        ],
        out_specs=[
            pl.BlockSpec((256, H * D), lambda i: (i, 0)),
            pl.BlockSpec((256, D), lambda i: (i, 0)),
        ],
        out_shape=[
            jax.ShapeDtypeStruct((NP, H * D), _f32),
            jax.ShapeDtypeStruct((NP, D), _f32),
        ],
    )(part, b.reshape(1, D), W, A)


def _fin_body(p_ref, b_ref, o_ref):
    o_ref[...] = _combine(p_ref, b_ref)


def _fin(part, b):
    return pl.pallas_call(
        _fin_body,
        grid=(NP // 256,),
        in_specs=[
            pl.BlockSpec((2, 3, 256, D), lambda i: (0, 0, i, 0)),
            pl.BlockSpec((1, D), lambda i: (0, 0)),
        ],
        out_specs=pl.BlockSpec((256, D), lambda i: (i, 0)),
        out_shape=jax.ShapeDtypeStruct((NP, D), _f32),
    )(part, b.reshape(1, D))


DW = 144

def _sc_edge_body(hrows, adT, srcv, dstv, part,
                  num_sh, adl, srcb, dstb, rowb, obuf, sem):
    c = lax.axis_index("c")
    s = lax.axis_index("s")
    nbase = s * NPT
    ebase = s * EPT
    zero16 = jnp.zeros((16,), _f32)
    zi16 = jnp.zeros((16,), _i32)
    c128 = jnp.full((16,), D, _i32)
    lane = lax.iota(_i32, 16)

    for hh in range(3):
        head = c * 3 + hh

        pltpu.sync_copy(adT.at[head], adl)

        def zero_rows(i, _):
            for j in range(DW // 16):
                rowb[i, pl.ds(j * 16, 16)] = zero16
            return 0

        lax.fori_loop(0, CHUNK, zero_rows, 0)
        for q in range(NPT // CHUNK):
            pltpu.sync_copy(rowb, num_sh.at[pl.ds(nbase + q * CHUNK, CHUNK), :])
        plsc.subcore_barrier()

        def chunk_step(k, _):
            base = ebase + k * CHUNK
            pltpu.sync_copy(srcv.at[pl.ds(base, CHUNK)], srcb)
            pltpu.sync_copy(dstv.at[pl.ds(base, CHUNK)], dstb)
            for g in range(CHUNK // 16):
                sv = srcb[pl.ds(g * 16, 16)]
                srcb[pl.ds(g * 16, 16)] = sv * H + head
            pltpu.async_copy(hrows.at[srcb], rowb, sem).wait()
            for g in range(CHUNK // 16):
                dv = dstb[pl.ds(g * 16, 16)]
                asv = plsc.load_gather(rowb, [g * 16 + lane, c128])
                a = asv + plsc.load_gather(adl, [dv])
                a = jnp.maximum(a, 0.2 * a)
                ex = jnp.exp(a)
                plsc.store_scatter(rowb, [g * 16 + lane, c128], ex)

            def scale_e(e, _):
                spl = plsc.load_gather(rowb, [zi16 + e, c128])
                for j in range(D // 16):
                    rowb[e, pl.ds(j * 16, 16)] = rowb[e, pl.ds(j * 16, 16)] * spl
                return 0

            lax.fori_loop(0, CHUNK, scale_e, 0)
            pltpu.sync_copy(rowb, num_sh.at[dstb], add=True)
            return 0

        lax.fori_loop(0, NCHUNK, chunk_step, 0)
        plsc.subcore_barrier()

        for q in range(NPT // CHUNK):
            pltpu.sync_copy(num_sh.at[pl.ds(nbase + q * CHUNK, CHUNK), :], rowb)

            def div_e(e, _):
                d = plsc.load_gather(rowb, [zi16 + e, c128])
                r = 1.0 / (d + 1e-16)
                for j in range(D // 16):
                    obuf[e, pl.ds(j * 16, 16)] = rowb[e, pl.ds(j * 16, 16)] * r
                return 0

            lax.fori_loop(0, CHUNK, div_e, 0)
            pltpu.sync_copy(obuf,
                            part.at[c, hh, pl.ds(nbase + q * CHUNK, CHUNK), :])


@functools.partial(
    pl.kernel,
    out_type=jax.ShapeDtypeStruct((2, 3, NP, D), _f32),
    mesh=plsc.VectorSubcoreMesh(core_axis_name="c", subcore_axis_name="s"),
    compiler_params=pltpu.CompilerParams(needs_layout_passes=False,
                                         use_tc_tiling_on_sc=False),
    scratch_types=[
        pltpu.VMEM_SHARED((NP, DW), _f32),
        pltpu.VMEM((NP,), _f32),
        pltpu.VMEM((CHUNK,), _i32),
        pltpu.VMEM((CHUNK,), _i32),
        pltpu.VMEM((CHUNK, DW), _f32),
        pltpu.VMEM((CHUNK, D), _f32),
        pltpu.SemaphoreType.DMA,
    ],
)
def _sc_edge(*refs):
    _sc_edge_body(*refs)


def _build_A(a_s, a_d):
    r = jnp.arange(H * D, dtype=_i32)
    A = jnp.zeros((H * D, D), _f32)
    A = A.at[r, r // D].set(a_s.reshape(H * D))
    A = A.at[r, 8 + r // D].set(a_d.reshape(H * D))
    return A


def kernel(x, edge_index, edge_attr, W0, a_src0, a_dst0, b0,
           W1, a_src1, a_dst1, b1):
    xp = jnp.zeros((NP, D), _f32).at[:N].set(x)
    loops = jnp.arange(N, dtype=_i32)
    src = jnp.concatenate([edge_index[0], loops,
                           jnp.zeros((EP - ETOT,), _i32)])
    dst = jnp.concatenate([edge_index[1], loops,
                           jnp.full((EP - ETOT,), N, _i32)])
    A0 = _build_A(a_src0, a_dst0)
    A1 = _build_A(a_src1, a_dst1)

    def rows144(h, al):
        return jnp.concatenate(
            [h.reshape(NP * H, D),
             al[:, 0:H].reshape(NP * H, 1),
             jnp.zeros((NP * H, DW - D - 1), _f32)], axis=1)

    h0, al0 = _mm1(xp, W0, A0)
    part0 = _sc_edge(rows144(h0, al0), al0[:, 8:8 + H].T, src, dst)

    h1, al1 = _mm2(part0, b0, W1, A1)
    part1 = _sc_edge(rows144(h1, al1), al1[:, 8:8 + H].T, src, dst)

    out = _fin(part1, b1)
    return out[:N]
